# baseline (speedup 1.0000x reference)
import functools

import jax
import jax.numpy as jnp
from jax import lax
from jax.experimental import pallas as pl
from jax.experimental.pallas import tpu as pltpu
from jax.experimental.pallas import tpu_sc as plsc

D = 2048

_info = plsc.get_sparse_core_info()
NC, NS, L = _info.num_cores, _info.num_subcores, _info.num_lanes
NW = NC * NS

B = 4 * 4096
B_PER_W = B // NW
CH = 8
NBUF = 7
AHEAD = 5
N_CHUNKS = B_PER_W // CH


def _make_gather():
    mesh = plsc.VectorSubcoreMesh(core_axis_name="c", subcore_axis_name="s")

    @functools.partial(
        pl.kernel,
        mesh=mesh,
        out_type=jax.ShapeDtypeStruct((B, D), jnp.float32),
        scratch_types=[
            pltpu.VMEM((B_PER_W,), jnp.int32),
        ]
        + [pltpu.VMEM((CH, D), jnp.float32) for _ in range(NBUF)]
        + [pltpu.SemaphoreType.DMA for _ in range(2 * NBUF)],
    )
    def k(table_hbm, idx_hbm, out_hbm, idx_v, *bufs_and_sems):
        bufs = bufs_and_sems[:NBUF]
        gsem = bufs_and_sems[NBUF:2 * NBUF]
        wsem = bufs_and_sems[2 * NBUF:]
        wid = lax.axis_index("s") * NC + lax.axis_index("c")
        base = wid * B_PER_W
        pltpu.sync_copy(idx_hbm.at[pl.ds(base, B_PER_W)], idx_v)

        def fire_gather(c, j):
            pltpu.async_copy(
                table_hbm.at[idx_v.at[pl.ds(c * CH, CH)]], bufs[j], gsem[j]
            )

        def wait_gather(c, j):
            pltpu.make_async_copy(
                table_hbm.at[idx_v.at[pl.ds(c * CH, CH)]], bufs[j], gsem[j]
            ).wait()

        def fire_write(c, j):
            pltpu.async_copy(
                bufs[j], out_hbm.at[pl.ds(base + c * CH, CH)], wsem[j]
            )

        def wait_write(c, j):
            pltpu.make_async_copy(
                bufs[j], out_hbm.at[pl.ds(base + c * CH, CH)], wsem[j]
            ).wait()

        for c in range(AHEAD):
            fire_gather(c, c % NBUF)

        def step(c, j):
            wait_gather(c, j)
            fire_write(c, j)
            if c + AHEAD < N_CHUNKS:
                if c - 2 >= 0:
                    wait_write(c - 2, (c - 2) % NBUF)
                fire_gather(c + AHEAD, (c + AHEAD) % NBUF)

        for j in range(NBUF):
            step(j, j)

        def body(r, carry):
            c0 = NBUF * r
            for j in range(NBUF):
                c = c0 + j
                wait_gather(c, j)
                fire_write(c, j)
                wait_write(c - 2, (j - 2) % NBUF)
                fire_gather(c + AHEAD, (j + AHEAD) % NBUF)
            return carry

        n_full_rounds = 8
        lax.fori_loop(1, n_full_rounds, body, 0, unroll=False)

        t0 = n_full_rounds * NBUF
        for c in range(t0, N_CHUNKS):
            step(c, c % NBUF)

        for c in range(N_CHUNKS - AHEAD - 2, N_CHUNKS):
            if c >= 0:
                wait_write(c, c % NBUF)

    return k


_gather = _make_gather()


def kernel(x, weight):
    idx = x.reshape(B).astype(jnp.int32)
    out = _gather(weight, idx)
    return out.reshape(x.shape + (D,))
